# Initial kernel scaffold; baseline (speedup 1.0000x reference)
import jax
import jax.numpy as jnp
from jax.experimental import pallas as pl


def kernel(token_ids, table_VE, W_EH):
    raise NotImplementedError("write your pallas kernel here")


# speedup vs baseline: 13.6106x; 13.6106x over previous
import functools

import jax
import jax.numpy as jnp
from jax.experimental import pallas as pl
from jax.experimental.pallas import tpu as pltpu
from jax.experimental.pallas import tpu_sc as plsc

RANK = 16
EMBED = 64
GROUP = 16
GATHER_WINDOW = 128
MM_BLOCK_M = 1024


def _sc_gather(table_VE, idx_flat):
    n = idx_flat.shape[0]
    idx2d = idx_flat.reshape(1, n)
    mesh = plsc.VectorSubcoreMesh(core_axis_name="core",
                                  subcore_axis_name="subcore")

    @functools.partial(
        pl.kernel,
        out_type=jax.ShapeDtypeStruct((n, RANK), jnp.float32),
        mesh=mesh,
        compiler_params=pltpu.CompilerParams(use_tc_tiling_on_sc=False),
    )
    def gather_kernel(table_hbm, i_hbm, o_hbm):
        def body(i_vmem, o_vmem):
            pltpu.sync_copy(table_hbm.at[i_vmem.at[0]], o_vmem)

        pltpu.emit_pipeline(
            body,
            grid=(n // GATHER_WINDOW,),
            in_specs=[pl.BlockSpec((1, GATHER_WINDOW),
                                   index_map=lambda i: (0, i))],
            out_specs=[pl.BlockSpec((GATHER_WINDOW, RANK),
                                    index_map=lambda i: (i, 0))],
            core_axis_name=("core", "subcore"),
            dimension_semantics=(pltpu.PARALLEL,),
        )(i_hbm, o_hbm)

    return gather_kernel(table_VE, idx2d)


def _mm_body(x_ref, w_ref, o_ref):
    o_ref[...] = jnp.dot(x_ref[...].astype(jnp.bfloat16), w_ref[...],
                         preferred_element_type=jnp.float32)


def _tc_project(emb_grouped, w_block):
    m = emb_grouped.shape[0]
    k = GROUP * RANK
    nn = GROUP * EMBED
    return pl.pallas_call(
        _mm_body,
        grid=(m // MM_BLOCK_M,),
        in_specs=[
            pl.BlockSpec((MM_BLOCK_M, k), lambda i: (i, 0)),
            pl.BlockSpec((k, nn), lambda i: (0, 0)),
        ],
        out_specs=pl.BlockSpec((MM_BLOCK_M, nn), lambda i: (i, 0)),
        out_shape=jax.ShapeDtypeStruct((m, nn), jnp.float32),
    )(emb_grouped, w_block)


def kernel(token_ids, table_VE, W_EH):
    batch, hist = token_ids.shape
    n = batch * hist
    idx_flat = token_ids.reshape(n).astype(jnp.int32)

    emb = _sc_gather(table_VE, idx_flat)
    emb_grouped = emb.reshape(n // GROUP, GROUP * RANK)

    w_block = jnp.kron(jnp.eye(GROUP, dtype=jnp.bfloat16),
                       W_EH.T.astype(jnp.bfloat16))

    out = _tc_project(emb_grouped, w_block)
    return out.reshape(batch, hist, EMBED)
